# baseline (speedup 1.0000x reference)
import functools

import jax
import jax.numpy as jnp
from jax import lax
from jax.experimental import pallas as pl
from jax.experimental.pallas import tpu as pltpu
from jax.experimental.pallas import tpu_sc as plsc

NUM_USERS = 10000
NUM_ITEMS = 40000
NN = NUM_USERS + NUM_ITEMS
D = 64
B = 4096
S = 2 * B
E = 800000

NC = 2
NS = 16
NW = NC * NS

CHUNK = 1280
NGRP = CHUNK // 16
MAXC = 20
RB = 128
NSTR = 4
SUB = NSTR * RB
NBUF = 1024
CBUF = CHUNK + SUB + 32
ACC_ROWS = S + 16
PAD_SLOT = S


def _sc_body(ego_hbm, nodes_hbm, src_hbm, dst_hbm, val_hbm,
             part_hbm, egob_hbm,
             pos_v, nodes_v, src_v, dst_v, val_v,
             csrc_v, cval_v, cp_v, rows_v, stage_v, acc_sh,
             csem0, csem1, gsem0, gsem1, ssem0, ssem1):
  cid = lax.axis_index("c")
  sid = lax.axis_index("s")
  wid = sid * NC + cid

  def memset_body(i, _):
    pos_v[pl.ds(i * 16, 16)] = jnp.full((16,), -1, jnp.int32)
    return 0
  lax.fori_loop(0, NN // 16, memset_body, 0)

  def nodes_chunk_body(c, _):
    pltpu.sync_copy(nodes_hbm.at[pl.ds(c * NBUF, NBUF)], nodes_v)

    def posset_body(g, _):
      n16 = nodes_v[pl.ds(g * 16, 16)]
      s16 = jnp.full((16,), c * NBUF + g * 16, jnp.int32) + lax.iota(jnp.int32, 16)
      plsc.store_scatter(pos_v, [n16], s16)
      return 0
    lax.fori_loop(0, NBUF // 16, posset_body, 0)
    return 0
  lax.fori_loop(0, S // NBUF, nodes_chunk_body, 0)

  def zero_rows_body(i, _):
    for q in range(D // 16):
      rows_v[i, pl.ds(q * 16, 16)] = jnp.zeros((16,), jnp.float32)
    return 0
  lax.fori_loop(0, SUB, zero_rows_body, 0)

  @pl.when(sid == 0)
  def _():
    def acc_zero_body(i, _):
      pltpu.sync_copy(rows_v, acc_sh.at[pl.ds(i * SUB, SUB)])
      return 0
    lax.fori_loop(0, S // SUB, acc_zero_body, 0)

  plsc.subcore_barrier()

  nchunks = jnp.where(wid < 17, 20, 19)
  c0 = wid * 19 + jnp.minimum(wid, 17)
  csems = (csem0, csem1)

  def issue_chunk(ci, b):
    e0 = (c0 + ci) * CHUNK
    pltpu.async_copy(src_hbm.at[pl.ds(e0, CHUNK)], src_v.at[b], csems[b])
    pltpu.async_copy(dst_hbm.at[pl.ds(e0, CHUNK)], dst_v.at[b], csems[b])
    pltpu.async_copy(val_hbm.at[pl.ds(e0, CHUNK)], val_v.at[b], csems[b])

  def wait_chunk(ci, b):
    e0 = (c0 + ci) * CHUNK
    pltpu.make_async_copy(src_hbm.at[pl.ds(e0, CHUNK)], src_v.at[b], csems[b]).wait()
    pltpu.make_async_copy(dst_hbm.at[pl.ds(e0, CHUNK)], dst_v.at[b], csems[b]).wait()
    pltpu.make_async_copy(val_hbm.at[pl.ds(e0, CHUNK)], val_v.at[b], csems[b]).wait()

  def process_chunk(ci, b):
    def filt_body(g, cnt):
      dv = dst_v[b, pl.ds(g * 16, 16)]
      pv = plsc.load_gather(pos_v, [dv])
      m = pv >= 0
      sv = src_v[b, pl.ds(g * 16, 16)]
      vv = val_v[b, pl.ds(g * 16, 16)]
      plsc.store_compressed(csrc_v.at[pl.ds(cnt, 16)], sv, mask=m)
      plsc.store_compressed(cval_v.at[pl.ds(cnt, 16)], vv, mask=m)
      plsc.store_compressed(cp_v.at[pl.ds(cnt, 16)], pv, mask=m)
      return cnt + jnp.sum(m.astype(jnp.int32))
    cnt = lax.fori_loop(0, NGRP, filt_body, jnp.int32(0))

    def pad_body(t, _):
      idxv = jnp.full((16,), cnt + t * 16, jnp.int32) + lax.iota(jnp.int32, 16)
      plsc.store_scatter(csrc_v, [idxv], jnp.zeros((16,), jnp.int32))
      plsc.store_scatter(cval_v, [idxv], jnp.zeros((16,), jnp.float32))
      plsc.store_scatter(cp_v, [idxv], jnp.full((16,), PAD_SLOT, jnp.int32))
      return 0
    lax.fori_loop(0, SUB // 16 + 1, pad_body, 0)

    nsb = (cnt + (SUB - 1)) // SUB

    def sub_body(sb, _):
      off = sb * SUB
      for t in range(NSTR):
        pltpu.async_copy(ego_hbm.at[csrc_v.at[pl.ds(off + t * RB, RB)]],
                         rows_v.at[pl.ds(t * RB, RB)], gsem0)
      for t in range(NSTR):
        pltpu.make_async_copy(ego_hbm.at[csrc_v.at[pl.ds(off + t * RB, RB)]],
                              rows_v.at[pl.ds(t * RB, RB)], gsem0).wait()

      def scale_body(j4, _):
        for r in range(4):
          j = j4 * 4 + r
          bv = plsc.load_gather(cval_v, [jnp.full((16,), off + j, jnp.int32)])
          for q in range(D // 16):
            rows_v[j, pl.ds(q * 16, 16)] = rows_v[j, pl.ds(q * 16, 16)] * bv
        return 0
      lax.fori_loop(0, SUB // 4, scale_body, 0)

      for t in range(NSTR):
        for k in range(RB // 16):
          stage_v[t, pl.ds(k * 16, 16)] = cp_v[pl.ds(off + t * RB + k * 16, 16)]
      for t in range(NSTR):
        pltpu.async_copy(rows_v.at[pl.ds(t * RB, RB)],
                         acc_sh.at[stage_v.at[t]], ssem0, add=True)
      for t in range(NSTR):
        pltpu.make_async_copy(rows_v.at[pl.ds(t * RB, RB)],
                              acc_sh.at[stage_v.at[t]], ssem0).wait()
      return 0
    lax.fori_loop(0, nsb, sub_body, 0)

  issue_chunk(0, 0)
  for ci in range(MAXC):
    b = ci % 2

    def body_ci(ci=ci, b=b):
      wait_chunk(ci, b)
      if ci + 1 < MAXC - 1:
        issue_chunk(ci + 1, 1 - b)
      elif ci + 1 == MAXC - 1:
        @pl.when(wid < 17)
        def _():
          issue_chunk(ci + 1, 1 - b)
      process_chunk(ci, b)

    if ci < MAXC - 1:
      body_ci()
    else:
      pl.when(wid < 17)(body_ci)

  plsc.subcore_barrier()

  slot0 = sid * (S // NS)

  base = slot0
  pltpu.sync_copy(nodes_hbm.at[pl.ds(base, SUB)], nodes_v.at[pl.ds(0, SUB)])

  def pos_lookup_body(i, _):
    n16 = nodes_v[pl.ds(i * 16, 16)]
    p16 = plsc.load_gather(pos_v, [n16])
    cp_v[pl.ds(i * 16, 16)] = p16
    return 0
  lax.fori_loop(0, SUB // 16, pos_lookup_body, 0)

  for t in range(NSTR):
    pltpu.async_copy(acc_sh.at[cp_v.at[pl.ds(t * RB, RB)]],
                     rows_v.at[pl.ds(t * RB, RB)], gsem0)
  for t in range(NSTR):
    pltpu.make_async_copy(acc_sh.at[cp_v.at[pl.ds(t * RB, RB)]],
                          rows_v.at[pl.ds(t * RB, RB)], gsem0).wait()
  pltpu.sync_copy(rows_v, part_hbm.at[cid].at[pl.ds(base, SUB)])

  @pl.when(cid == 1)
  def _():
    for t in range(NSTR):
      pltpu.async_copy(ego_hbm.at[nodes_v.at[pl.ds(t * RB, RB)]],
                       rows_v.at[pl.ds(t * RB, RB)], gsem1)
    for t in range(NSTR):
      pltpu.make_async_copy(ego_hbm.at[nodes_v.at[pl.ds(t * RB, RB)]],
                            rows_v.at[pl.ds(t * RB, RB)], gsem1).wait()
    pltpu.sync_copy(rows_v, egob_hbm.at[pl.ds(base, SUB)])


@jax.jit
def _sc_spmm(ego, nodes, esrc, edst, eval_):
  mesh = plsc.VectorSubcoreMesh(core_axis_name="c", subcore_axis_name="s")
  fn = pl.kernel(
      _sc_body,
      out_type=(
          jax.ShapeDtypeStruct((NC, S, D), jnp.float32),
          jax.ShapeDtypeStruct((S, D), jnp.float32),
      ),
      mesh=mesh,
      scratch_types=[
          pltpu.VMEM((NN,), jnp.int32),
          pltpu.VMEM((NBUF,), jnp.int32),
          pltpu.VMEM((2, CHUNK), jnp.int32),
          pltpu.VMEM((2, CHUNK), jnp.int32),
          pltpu.VMEM((2, CHUNK), jnp.float32),
          pltpu.VMEM((CBUF,), jnp.int32),
          pltpu.VMEM((CBUF,), jnp.float32),
          pltpu.VMEM((CBUF,), jnp.int32),
          pltpu.VMEM((SUB, D), jnp.float32),
          pltpu.VMEM((NSTR, RB), jnp.int32),
          pltpu.VMEM_SHARED((ACC_ROWS, D), jnp.float32),
          pltpu.SemaphoreType.DMA,
          pltpu.SemaphoreType.DMA,
          pltpu.SemaphoreType.DMA,
          pltpu.SemaphoreType.DMA,
          pltpu.SemaphoreType.DMA,
          pltpu.SemaphoreType.DMA,
      ],
      compiler_params=pltpu.CompilerParams(needs_layout_passes=False,
                                           use_tc_tiling_on_sc=False),
  )
  return fn(ego, nodes, esrc, edst, eval_)


def _tc_norm_body(part_ref, egob_ref, xn_ref, scal_ref):
  eg = egob_ref[...]
  z = part_ref[0] + part_ref[1] + 2.0 * eg
  nrm = jnp.sqrt(jnp.sum(z * z, axis=1, keepdims=True))
  zn = z / jnp.maximum(nrm, 1e-12)
  un = zn[:B]
  inn = zn[B:]
  xn_ref[0] = un
  xn_ref[1] = inn
  scal_ref[0] = jnp.mean(jnp.sum((un - inn) ** 2, axis=1))
  scal_ref[1] = 1e-4 * 0.5 * jnp.sum(eg * eg) / B


@jax.jit
def _tc_norm(part, egob):
  return pl.pallas_call(
      _tc_norm_body,
      out_shape=(
          jax.ShapeDtypeStruct((2, B, D), jnp.float32),
          jax.ShapeDtypeStruct((2,), jnp.float32),
      ),
      out_specs=(
          pl.BlockSpec(),
          pl.BlockSpec(memory_space=pltpu.MemorySpace.SMEM),
      ),
  )(part, egob)


TB = 512
NT = B // TB


def _tc_uni_body(scal_ref, a_ref, b_ref, out_ref, acc_ref):
  g = pl.program_id(0)
  i = pl.program_id(1)
  j = pl.program_id(2)

  @pl.when(jnp.logical_and(i == 0, j == 0))
  def _():
    acc_ref[g] = 0.0

  @pl.when(j >= i)
  def _():
    dots = lax.dot_general(a_ref[0], b_ref[0], (((1,), (1,)), ((), ())),
                           preferred_element_type=jnp.float32)
    sq = jnp.maximum(2.0 - 2.0 * dots, 0.0)
    ev = jnp.exp(-2.0 * sq)
    rid = lax.broadcasted_iota(jnp.int32, (TB, TB), 0)
    cidx = lax.broadcasted_iota(jnp.int32, (TB, TB), 1)
    keep = jnp.logical_or(j > i, cidx > rid)
    acc_ref[g] += jnp.sum(jnp.where(keep, ev, 0.0))

  @pl.when(jnp.logical_and(g == 1, jnp.logical_and(i == NT - 1, j == NT - 1)))
  def _():
    npairs = B * (B - 1) / 2.0
    uni = 0.5 * (jnp.log(acc_ref[0] / npairs) + jnp.log(acc_ref[1] / npairs))
    out_ref[0] = scal_ref[0]
    out_ref[1] = uni
    out_ref[2] = scal_ref[1]


@jax.jit
def _tc_uniform(scal, xn):
  return pl.pallas_call(
      _tc_uni_body,
      grid=(2, NT, NT),
      in_specs=[
          pl.BlockSpec((2,), lambda g, i, j: (0,),
                       memory_space=pltpu.MemorySpace.SMEM),
          pl.BlockSpec((1, TB, D), lambda g, i, j: (g, i, 0)),
          pl.BlockSpec((1, TB, D), lambda g, i, j: (g, j, 0)),
      ],
      out_specs=pl.BlockSpec((3,), lambda g, i, j: (0,),
                             memory_space=pltpu.MemorySpace.SMEM),
      out_shape=jax.ShapeDtypeStruct((3,), jnp.float32),
      scratch_shapes=[pltpu.SMEM((2,), jnp.float32)],
  )(scal, xn, xn)


def kernel(user, positive, negative, user_emb, item_emb, edge_src, edge_dst, edge_val):
  user = user.astype(jnp.int32)
  positive = positive.astype(jnp.int32)
  ego = jnp.concatenate([user_emb, item_emb], axis=0)
  nodes = jnp.concatenate([user, positive + NUM_USERS])
  part, egob = _sc_spmm(ego, nodes, edge_src.astype(jnp.int32),
                        edge_dst.astype(jnp.int32), edge_val)
  xn, scal = _tc_norm(part, egob)
  return _tc_uniform(scal, xn)


# speedup vs baseline: 7.1461x; 7.1461x over previous
import functools

import jax
import jax.numpy as jnp
from jax import lax
from jax.experimental import pallas as pl
from jax.experimental.pallas import tpu as pltpu
from jax.experimental.pallas import tpu_sc as plsc

NUM_USERS = 10000
NUM_ITEMS = 40000
NN = NUM_USERS + NUM_ITEMS
D = 64
B = 4096
S = 2 * B
E = 800000

NC = 2
NS = 16
NW = NC * NS

CHUNK = 2000
NGRP = CHUNK // 16
MAXC = 13
RB = 128
NSTR = 2
SUB = NSTR * RB
NBUF = 1024
CBUF = CHUNK + SUB + 48
ACC_ROWS = S + 16
PAD_SLOT = S


def _sc_body(ego_hbm, nodes_hbm, src_hbm, dst_hbm, val_hbm,
             part_hbm, egob_hbm,
             pos_v, nodes_v, src_v, dst_v, val_v,
             csrc_v, cval_v, cp_v, rows_v, stage_v, acc_sh,
             csem0, csem1, gsem0, gsem1, ssem0, ssem1):
  cid = lax.axis_index("c")
  sid = lax.axis_index("s")
  wid = sid * NC + cid

  def memset_body(i, _):
    pos_v[pl.ds(i * 16, 16)] = jnp.full((16,), -1, jnp.int32)
    return 0
  lax.fori_loop(0, NN // 16, memset_body, 0)

  def nodes_chunk_body(c, _):
    pltpu.sync_copy(nodes_hbm.at[pl.ds(c * NBUF, NBUF)], nodes_v)

    def posset_body(g, _):
      n16 = nodes_v[pl.ds(g * 16, 16)]
      s16 = jnp.full((16,), c * NBUF + g * 16, jnp.int32) + lax.iota(jnp.int32, 16)
      plsc.store_scatter(pos_v, [n16], s16)
      return 0
    lax.fori_loop(0, NBUF // 16, posset_body, 0)
    return 0
  lax.fori_loop(0, S // NBUF, nodes_chunk_body, 0)

  def zero_rows_body(i, _):
    for q in range(D // 16):
      rows_v[i, pl.ds(q * 16, 16)] = jnp.zeros((16,), jnp.float32)
    return 0
  lax.fori_loop(0, SUB, zero_rows_body, 0)

  @pl.when(sid == 0)
  def _():
    def acc_zero_body(i, _):
      pltpu.sync_copy(rows_v, acc_sh.at[pl.ds(i * SUB, SUB)])
      return 0
    lax.fori_loop(0, S // SUB, acc_zero_body, 0)

  plsc.subcore_barrier()

  nchunks = jnp.where(wid < 16, 13, 12)
  c0 = wid * 12 + jnp.minimum(wid, 16)
  csems = (csem0, csem1)

  def chunk_e0(ci):
    return jnp.minimum((c0 + ci) * CHUNK, E - CHUNK)

  def issue_chunk(ci, b):
    e0 = chunk_e0(ci)
    pltpu.async_copy(src_hbm.at[pl.ds(e0, CHUNK)], src_v.at[b], csems[b])
    pltpu.async_copy(dst_hbm.at[pl.ds(e0, CHUNK)], dst_v.at[b], csems[b])
    pltpu.async_copy(val_hbm.at[pl.ds(e0, CHUNK)], val_v.at[b], csems[b])

  def wait_chunk(ci, b):
    e0 = chunk_e0(ci)
    pltpu.make_async_copy(src_hbm.at[pl.ds(e0, CHUNK)], src_v.at[b], csems[b]).wait()
    pltpu.make_async_copy(dst_hbm.at[pl.ds(e0, CHUNK)], dst_v.at[b], csems[b]).wait()
    pltpu.make_async_copy(val_hbm.at[pl.ds(e0, CHUNK)], val_v.at[b], csems[b]).wait()

  def flush_batch(off):
    for t in range(NSTR):
      pltpu.async_copy(ego_hbm.at[csrc_v.at[pl.ds(off + t * RB, RB)]],
                       rows_v.at[pl.ds(t * RB, RB)], gsem0)
    for t in range(NSTR):
      pltpu.make_async_copy(ego_hbm.at[csrc_v.at[pl.ds(off + t * RB, RB)]],
                            rows_v.at[pl.ds(t * RB, RB)], gsem0).wait()

    def scale_body(j4, _):
      for r in range(4):
        j = j4 * 4 + r
        bv = plsc.load_gather(cval_v, [jnp.full((16,), off + j, jnp.int32)])
        for q in range(D // 16):
          rows_v[j, pl.ds(q * 16, 16)] = rows_v[j, pl.ds(q * 16, 16)] * bv
      return 0
    lax.fori_loop(0, SUB // 4, scale_body, 0)

    for t in range(NSTR):
      for k in range(RB // 16):
        stage_v[t, pl.ds(k * 16, 16)] = cp_v[pl.ds(off + t * RB + k * 16, 16)]
    for t in range(NSTR):
      pltpu.async_copy(rows_v.at[pl.ds(t * RB, RB)],
                       acc_sh.at[stage_v.at[t]], ssem0, add=True)
    for t in range(NSTR):
      pltpu.make_async_copy(rows_v.at[pl.ds(t * RB, RB)],
                            acc_sh.at[stage_v.at[t]], ssem0).wait()

  def process_chunk(ci, b, cnt):
    live = ci < nchunks

    def filt_body(g, cnt):
      dv = dst_v[b, pl.ds(g * 16, 16)]
      pv = plsc.load_gather(pos_v, [dv])
      m = jnp.logical_and(pv >= 0, live)
      sv = src_v[b, pl.ds(g * 16, 16)]
      vv = val_v[b, pl.ds(g * 16, 16)]
      plsc.store_compressed(csrc_v.at[pl.ds(cnt, 16)], sv, mask=m)
      plsc.store_compressed(cval_v.at[pl.ds(cnt, 16)], vv, mask=m)
      plsc.store_compressed(cp_v.at[pl.ds(cnt, 16)], pv, mask=m)
      return cnt + jnp.sum(m.astype(jnp.int32))
    cnt = lax.fori_loop(0, NGRP, filt_body, cnt)

    nfl = cnt // SUB

    def fl_body(f, _):
      flush_batch(f * SUB)
      return 0
    lax.fori_loop(0, nfl, fl_body, 0)

    rem = cnt - nfl * SUB

    @pl.when(nfl > 0)
    def _():
      def move_body(k, _):
        srcoff = nfl * SUB + k * 16
        csrc_v[pl.ds(k * 16, 16)] = csrc_v[pl.ds(srcoff, 16)]
        cval_v[pl.ds(k * 16, 16)] = cval_v[pl.ds(srcoff, 16)]
        cp_v[pl.ds(k * 16, 16)] = cp_v[pl.ds(srcoff, 16)]
        return 0
      lax.fori_loop(0, (rem + 15) // 16, move_body, 0)
    return rem

  cnt = jnp.int32(0)
  issue_chunk(0, 0)
  for ci in range(MAXC):
    b = ci % 2
    wait_chunk(ci, b)
    if ci + 1 < MAXC:
      issue_chunk(ci + 1, 1 - b)
    cnt = process_chunk(ci, b, cnt)

  def pad_body(t, _):
    idxv = jnp.full((16,), cnt + t * 16, jnp.int32) + lax.iota(jnp.int32, 16)
    plsc.store_scatter(csrc_v, [idxv], jnp.zeros((16,), jnp.int32))
    plsc.store_scatter(cval_v, [idxv], jnp.zeros((16,), jnp.float32))
    plsc.store_scatter(cp_v, [idxv], jnp.full((16,), PAD_SLOT, jnp.int32))
    return 0
  lax.fori_loop(0, SUB // 16 + 1, pad_body, 0)

  @pl.when(cnt > 0)
  def _():
    flush_batch(0)

  plsc.subcore_barrier()

  slot0 = sid * (S // NS)

  def redist_body(sb, _):
    base = slot0 + sb * SUB
    pltpu.sync_copy(nodes_hbm.at[pl.ds(base, SUB)], nodes_v.at[pl.ds(0, SUB)])

    def pos_lookup_body(i, _):
      n16 = nodes_v[pl.ds(i * 16, 16)]
      p16 = plsc.load_gather(pos_v, [n16])
      cp_v[pl.ds(i * 16, 16)] = p16
      return 0
    lax.fori_loop(0, SUB // 16, pos_lookup_body, 0)

    for t in range(NSTR):
      pltpu.async_copy(acc_sh.at[cp_v.at[pl.ds(t * RB, RB)]],
                       rows_v.at[pl.ds(t * RB, RB)], gsem0)
    for t in range(NSTR):
      pltpu.make_async_copy(acc_sh.at[cp_v.at[pl.ds(t * RB, RB)]],
                            rows_v.at[pl.ds(t * RB, RB)], gsem0).wait()
    pltpu.sync_copy(rows_v, part_hbm.at[cid].at[pl.ds(base, SUB)])

    @pl.when(cid == 1)
    def _():
      for t in range(NSTR):
        pltpu.async_copy(ego_hbm.at[nodes_v.at[pl.ds(t * RB, RB)]],
                         rows_v.at[pl.ds(t * RB, RB)], gsem1)
      for t in range(NSTR):
        pltpu.make_async_copy(ego_hbm.at[nodes_v.at[pl.ds(t * RB, RB)]],
                              rows_v.at[pl.ds(t * RB, RB)], gsem1).wait()
      pltpu.sync_copy(rows_v, egob_hbm.at[pl.ds(base, SUB)])
    return 0
  lax.fori_loop(0, (S // NS) // SUB, redist_body, 0)


@jax.jit
def _sc_spmm(ego, nodes, esrc, edst, eval_):
  mesh = plsc.VectorSubcoreMesh(core_axis_name="c", subcore_axis_name="s")
  fn = pl.kernel(
      _sc_body,
      out_type=(
          jax.ShapeDtypeStruct((NC, S, D), jnp.float32),
          jax.ShapeDtypeStruct((S, D), jnp.float32),
      ),
      mesh=mesh,
      scratch_types=[
          pltpu.VMEM((NN,), jnp.int32),
          pltpu.VMEM((NBUF,), jnp.int32),
          pltpu.VMEM((2, CHUNK), jnp.int32),
          pltpu.VMEM((2, CHUNK), jnp.int32),
          pltpu.VMEM((2, CHUNK), jnp.float32),
          pltpu.VMEM((CBUF,), jnp.int32),
          pltpu.VMEM((CBUF,), jnp.float32),
          pltpu.VMEM((CBUF,), jnp.int32),
          pltpu.VMEM((SUB, D), jnp.float32),
          pltpu.VMEM((NSTR, RB), jnp.int32),
          pltpu.VMEM_SHARED((ACC_ROWS, D), jnp.float32),
          pltpu.SemaphoreType.DMA,
          pltpu.SemaphoreType.DMA,
          pltpu.SemaphoreType.DMA,
          pltpu.SemaphoreType.DMA,
          pltpu.SemaphoreType.DMA,
          pltpu.SemaphoreType.DMA,
      ],
      compiler_params=pltpu.CompilerParams(needs_layout_passes=False,
                                           use_tc_tiling_on_sc=False),
  )
  return fn(ego, nodes, esrc, edst, eval_)


def _tc_norm_body(part_ref, egob_ref, xn_ref, scal_ref):
  eg = egob_ref[...]
  z = part_ref[0] + part_ref[1] + 2.0 * eg
  nrm = jnp.sqrt(jnp.sum(z * z, axis=1, keepdims=True))
  zn = z / jnp.maximum(nrm, 1e-12)
  un = zn[:B]
  inn = zn[B:]
  xn_ref[0] = un
  xn_ref[1] = inn
  scal_ref[0] = jnp.mean(jnp.sum((un - inn) ** 2, axis=1))
  scal_ref[1] = 1e-4 * 0.5 * jnp.sum(eg * eg) / B


@jax.jit
def _tc_norm(part, egob):
  return pl.pallas_call(
      _tc_norm_body,
      out_shape=(
          jax.ShapeDtypeStruct((2, B, D), jnp.float32),
          jax.ShapeDtypeStruct((2,), jnp.float32),
      ),
      out_specs=(
          pl.BlockSpec(),
          pl.BlockSpec(memory_space=pltpu.MemorySpace.SMEM),
      ),
  )(part, egob)


TB = 512
NT = B // TB


def _tc_uni_body(scal_ref, a_ref, b_ref, out_ref, acc_ref):
  g = pl.program_id(0)
  i = pl.program_id(1)
  j = pl.program_id(2)

  @pl.when(jnp.logical_and(i == 0, j == 0))
  def _():
    acc_ref[g] = 0.0

  @pl.when(j >= i)
  def _():
    dots = lax.dot_general(a_ref[0], b_ref[0], (((1,), (1,)), ((), ())),
                           preferred_element_type=jnp.float32)
    sq = jnp.maximum(2.0 - 2.0 * dots, 0.0)
    ev = jnp.exp(-2.0 * sq)
    rid = lax.broadcasted_iota(jnp.int32, (TB, TB), 0)
    cidx = lax.broadcasted_iota(jnp.int32, (TB, TB), 1)
    keep = jnp.logical_or(j > i, cidx > rid)
    acc_ref[g] += jnp.sum(jnp.where(keep, ev, 0.0))

  @pl.when(jnp.logical_and(g == 1, jnp.logical_and(i == NT - 1, j == NT - 1)))
  def _():
    npairs = B * (B - 1) / 2.0
    uni = 0.5 * (jnp.log(acc_ref[0] / npairs) + jnp.log(acc_ref[1] / npairs))
    out_ref[0] = scal_ref[0]
    out_ref[1] = uni
    out_ref[2] = scal_ref[1]


@jax.jit
def _tc_uniform(scal, xn):
  return pl.pallas_call(
      _tc_uni_body,
      grid=(2, NT, NT),
      in_specs=[
          pl.BlockSpec((2,), lambda g, i, j: (0,),
                       memory_space=pltpu.MemorySpace.SMEM),
          pl.BlockSpec((1, TB, D), lambda g, i, j: (g, i, 0)),
          pl.BlockSpec((1, TB, D), lambda g, i, j: (g, j, 0)),
      ],
      out_specs=pl.BlockSpec((3,), lambda g, i, j: (0,),
                             memory_space=pltpu.MemorySpace.SMEM),
      out_shape=jax.ShapeDtypeStruct((3,), jnp.float32),
      scratch_shapes=[pltpu.SMEM((2,), jnp.float32)],
  )(scal, xn, xn)


def kernel(user, positive, negative, user_emb, item_emb, edge_src, edge_dst, edge_val):
  user = user.astype(jnp.int32)
  positive = positive.astype(jnp.int32)
  ego = jnp.concatenate([user_emb, item_emb], axis=0)
  nodes = jnp.concatenate([user, positive + NUM_USERS])
  part, egob = _sc_spmm(ego, nodes, edge_src.astype(jnp.int32),
                        edge_dst.astype(jnp.int32), edge_val)
  xn, scal = _tc_norm(part, egob)
  return _tc_uniform(scal, xn)
